# baseline (speedup 1.0000x reference)
import functools

import jax
import jax.numpy as jnp
from jax.experimental import pallas as pl
from jax.experimental.pallas import tpu as pltpu

INPUT_DIM = 2048
OUTPUT_DIM = 2048
LORA_R = 64
NUM_EXPERTS = 8
LORA_ALPHA = 8.0
SCALING = LORA_ALPHA / LORA_R
ER = NUM_EXPERTS * LORA_R

BM = 1024


def _body(xb, wg, bg, a_t, b2, ob):
    xv = xb[...].astype(jnp.bfloat16)
    lgT = jax.lax.dot_general(wg[...], xv, (((1,), (1,)), ((), ())),
                              preferred_element_type=jnp.float32)
    lgT = lgT + bg[...][:, 0:1]
    row = jax.lax.broadcasted_iota(jnp.int32, (NUM_EXPERTS, BM), 0)
    mx = jnp.max(lgT, axis=0, keepdims=True)
    ex = jnp.exp(lgT - mx)
    sm = ex / jnp.sum(ex, axis=0, keepdims=True)
    m1 = jnp.max(sm, axis=0, keepdims=True)
    i1 = jnp.min(jnp.where(sm == m1, row, NUM_EXPERTS), axis=0, keepdims=True)
    sm2 = jnp.where(row == i1, -1.0, sm)
    m2 = jnp.max(sm2, axis=0, keepdims=True)
    i2 = jnp.min(jnp.where(sm2 == m2, row, NUM_EXPERTS), axis=0, keepdims=True)
    den = m1 + m2
    pack = jnp.concatenate(
        [i1.astype(jnp.float32), i2.astype(jnp.float32),
         m1 * (SCALING / den), m2 * (SCALING / den)],
        axis=0)
    packT = jnp.transpose(pack)
    i1c = packT[:, 0:1]
    i2c = packT[:, 1:2]
    w1c = packT[:, 2:3]
    w2c = packT[:, 3:4]
    ecol = (jax.lax.broadcasted_iota(jnp.int32, (BM, ER), 1) // LORA_R
            ).astype(jnp.float32)
    gates = jnp.where(ecol == i1c, w1c, 0.0) + jnp.where(ecol == i2c, w2c, 0.0)
    h = jax.lax.dot_general(xv, a_t[...],
                            (((1,), (1,)), ((), ())),
                            preferred_element_type=jnp.float32)
    hw = (h * gates).astype(jnp.bfloat16)
    ob[...] = jax.lax.dot_general(hw, b2[...], (((1,), (1,)), ((), ())),
                                  preferred_element_type=jnp.float32)


@jax.jit
def _run(flat, wg, bg, a_t, b2):
    n = flat.shape[0]
    grid = (n // BM,)
    return pl.pallas_call(
        _body,
        grid=grid,
        in_specs=[
            pl.BlockSpec((BM, INPUT_DIM), lambda i: (i, 0)),
            pl.BlockSpec((NUM_EXPERTS, INPUT_DIM), lambda i: (0, 0)),
            pl.BlockSpec((NUM_EXPERTS, 128), lambda i: (0, 0)),
            pl.BlockSpec((ER, INPUT_DIM), lambda i: (0, 0)),
            pl.BlockSpec((OUTPUT_DIM, ER), lambda i: (0, 0)),
        ],
        out_specs=pl.BlockSpec((BM, OUTPUT_DIM), lambda i: (i, 0)),
        out_shape=jax.ShapeDtypeStruct((n, OUTPUT_DIM), jnp.float32),
        compiler_params=pltpu.CompilerParams(
            dimension_semantics=("arbitrary",),
        ),
    )(flat, wg, bg, a_t, b2)


def kernel(x, W_gate, b_gate, A, B):
    flat = x.reshape(-1, x.shape[-1])
    wg = W_gate.astype(jnp.bfloat16)
    bg = jnp.broadcast_to(b_gate[:, None], (NUM_EXPERTS, 128))
    a_t = A.reshape(ER, INPUT_DIM).astype(jnp.bfloat16)
    b2 = B.transpose(1, 0, 2).reshape(OUTPUT_DIM, ER).astype(jnp.bfloat16)
    out = _run(flat, wg, bg, a_t, b2)
    return out.reshape(x.shape[:-1] + (OUTPUT_DIM,))


# speedup vs baseline: 1.0167x; 1.0167x over previous
import functools

import jax
import jax.numpy as jnp
from jax.experimental import pallas as pl
from jax.experimental.pallas import tpu as pltpu

INPUT_DIM = 2048
OUTPUT_DIM = 2048
LORA_R = 64
NUM_EXPERTS = 8
LORA_ALPHA = 8.0
SCALING = LORA_ALPHA / LORA_R
ER = NUM_EXPERTS * LORA_R

BM = 1024


def _body(xb, wg, bg, a_t, b2, ob):
    xv = xb[...].astype(jnp.bfloat16)
    lgT = jax.lax.dot_general(wg[...], xv, (((1,), (1,)), ((), ())),
                              preferred_element_type=jnp.float32)
    lgT = lgT + bg[...][:, 0:1]
    row = jax.lax.broadcasted_iota(jnp.int32, (NUM_EXPERTS, BM), 0)
    mx = jnp.max(lgT, axis=0, keepdims=True)
    ex = jnp.exp(lgT - mx)
    sm = ex / jnp.sum(ex, axis=0, keepdims=True)
    m1 = jnp.max(sm, axis=0, keepdims=True)
    i1 = jnp.min(jnp.where(sm == m1, row, NUM_EXPERTS), axis=0, keepdims=True)
    sm2 = jnp.where(row == i1, -1.0, sm)
    m2 = jnp.max(sm2, axis=0, keepdims=True)
    i2 = jnp.min(jnp.where(sm2 == m2, row, NUM_EXPERTS), axis=0, keepdims=True)
    den = m1 + m2
    pack = jnp.concatenate(
        [i1.astype(jnp.float32), i2.astype(jnp.float32),
         m1 * (SCALING / den), m2 * (SCALING / den)],
        axis=0)
    packT = jnp.transpose(pack)
    i1c = packT[:, 0:1]
    i2c = packT[:, 1:2]
    w1c = packT[:, 2:3]
    w2c = packT[:, 3:4]
    ecol = (jax.lax.broadcasted_iota(jnp.int32, (BM, ER), 1) // LORA_R
            ).astype(jnp.float32)
    gates = jnp.where(ecol == i1c, w1c, 0.0) + jnp.where(ecol == i2c, w2c, 0.0)
    h = jax.lax.dot_general(xv, a_t[...],
                            (((1,), (1,)), ((), ())),
                            preferred_element_type=jnp.float32)
    hw = (h * gates).astype(jnp.bfloat16)
    ob[...] = jnp.dot(hw, b2[...], preferred_element_type=jnp.float32)


@jax.jit
def _run(flat, wg, bg, a_t, b2):
    n = flat.shape[0]
    grid = (n // BM,)
    return pl.pallas_call(
        _body,
        grid=grid,
        in_specs=[
            pl.BlockSpec((BM, INPUT_DIM), lambda i: (i, 0)),
            pl.BlockSpec((NUM_EXPERTS, INPUT_DIM), lambda i: (0, 0)),
            pl.BlockSpec((NUM_EXPERTS, 128), lambda i: (0, 0)),
            pl.BlockSpec((ER, INPUT_DIM), lambda i: (0, 0)),
            pl.BlockSpec((ER, OUTPUT_DIM), lambda i: (0, 0)),
        ],
        out_specs=pl.BlockSpec((BM, OUTPUT_DIM), lambda i: (i, 0)),
        out_shape=jax.ShapeDtypeStruct((n, OUTPUT_DIM), jnp.float32),
        compiler_params=pltpu.CompilerParams(
            dimension_semantics=("arbitrary",),
        ),
    )(flat, wg, bg, a_t, b2)


def kernel(x, W_gate, b_gate, A, B):
    flat = x.reshape(-1, x.shape[-1])
    wg = W_gate.astype(jnp.bfloat16)
    bg = jnp.broadcast_to(b_gate[:, None], (NUM_EXPERTS, 128))
    a_t = A.reshape(ER, INPUT_DIM).astype(jnp.bfloat16)
    b2 = B.transpose(0, 2, 1).reshape(ER, OUTPUT_DIM).astype(jnp.bfloat16)
    out = _run(flat, wg, bg, a_t, b2)
    return out.reshape(x.shape[:-1] + (OUTPUT_DIM,))


# speedup vs baseline: 1.0504x; 1.0331x over previous
import functools

import jax
import jax.numpy as jnp
from jax.experimental import pallas as pl
from jax.experimental.pallas import tpu as pltpu

INPUT_DIM = 2048
OUTPUT_DIM = 2048
LORA_R = 64
NUM_EXPERTS = 8
LORA_ALPHA = 8.0
SCALING = LORA_ALPHA / LORA_R
ER = NUM_EXPERTS * LORA_R

BM = 1024


def _body(xb, wg, bg, ar, b2, ob, a_bf):
    @pl.when(pl.program_id(0) == 0)
    def _prep():
        a_bf[...] = ar[...].astype(jnp.bfloat16)

    xv = xb[...].astype(jnp.bfloat16)
    lgT = jax.lax.dot_general(wg[...].astype(jnp.bfloat16), xv,
                              (((1,), (1,)), ((), ())),
                              preferred_element_type=jnp.float32)
    lgT = lgT + bg[...]
    row = jax.lax.broadcasted_iota(jnp.int32, (NUM_EXPERTS, BM), 0)
    mx = jnp.max(lgT, axis=0, keepdims=True)
    ex = jnp.exp(lgT - mx)
    sm = ex / jnp.sum(ex, axis=0, keepdims=True)
    m1 = jnp.max(sm, axis=0, keepdims=True)
    i1 = jnp.min(jnp.where(sm == m1, row, NUM_EXPERTS), axis=0, keepdims=True)
    sm2 = jnp.where(row == i1, -1.0, sm)
    m2 = jnp.max(sm2, axis=0, keepdims=True)
    i2 = jnp.min(jnp.where(sm2 == m2, row, NUM_EXPERTS), axis=0, keepdims=True)
    den = m1 + m2
    pack = jnp.concatenate(
        [i1.astype(jnp.float32), i2.astype(jnp.float32),
         m1 * (SCALING / den), m2 * (SCALING / den)],
        axis=0)
    packT = jnp.transpose(pack)
    i1c = packT[:, 0:1]
    i2c = packT[:, 1:2]
    w1c = packT[:, 2:3]
    w2c = packT[:, 3:4]
    ecol = (jax.lax.broadcasted_iota(jnp.int32, (BM, ER), 1) // LORA_R
            ).astype(jnp.float32)
    gates = jnp.where(ecol == i1c, w1c, 0.0) + jnp.where(ecol == i2c, w2c, 0.0)
    h = jax.lax.dot_general(xv, a_bf[...],
                            (((1,), (1,)), ((), ())),
                            preferred_element_type=jnp.float32)
    hw = (h * gates).astype(jnp.bfloat16)
    ob[...] = jnp.dot(hw, b2[...], preferred_element_type=jnp.float32)


@jax.jit
def _run(flat, wg, bg, a_view, b2):
    n = flat.shape[0]
    grid = (n // BM,)
    return pl.pallas_call(
        _body,
        grid=grid,
        in_specs=[
            pl.BlockSpec((BM, INPUT_DIM), lambda i: (i, 0)),
            pl.BlockSpec((NUM_EXPERTS, INPUT_DIM), lambda i: (0, 0)),
            pl.BlockSpec((NUM_EXPERTS, 1), lambda i: (0, 0)),
            pl.BlockSpec((ER, INPUT_DIM), lambda i: (0, 0)),
            pl.BlockSpec((ER, OUTPUT_DIM), lambda i: (0, 0)),
        ],
        out_specs=pl.BlockSpec((BM, OUTPUT_DIM), lambda i: (i, 0)),
        out_shape=jax.ShapeDtypeStruct((n, OUTPUT_DIM), jnp.float32),
        scratch_shapes=[pltpu.VMEM((ER, INPUT_DIM), jnp.bfloat16)],
        compiler_params=pltpu.CompilerParams(
            dimension_semantics=("arbitrary",),
        ),
    )(flat, wg, bg, a_view, b2)


def kernel(x, W_gate, b_gate, A, B):
    flat = x.reshape(-1, x.shape[-1])
    bg = b_gate.reshape(NUM_EXPERTS, 1)
    a_view = A.reshape(ER, INPUT_DIM)
    b2 = B.transpose(0, 2, 1).reshape(ER, OUTPUT_DIM).astype(jnp.bfloat16)
    out = _run(flat, W_gate, bg, a_view, b2)
    return out.reshape(x.shape[:-1] + (OUTPUT_DIM,))


# speedup vs baseline: 1.0604x; 1.0095x over previous
import jax
import jax.numpy as jnp
from jax.experimental import pallas as pl
from jax.experimental.pallas import tpu as pltpu

INPUT_DIM = 2048
OUTPUT_DIM = 2048
LORA_R = 64
NUM_EXPERTS = 8
LORA_ALPHA = 8.0
SCALING = LORA_ALPHA / LORA_R
ER = NUM_EXPERTS * LORA_R

BM = 1024


def _body(xb, wg, bg, ar, b2, ob, a_bf):
    @pl.when(pl.program_id(0) == 0)
    def _prep():
        a_bf[...] = ar[...].astype(jnp.bfloat16)

    xv = xb[...].astype(jnp.bfloat16)
    lgT = jax.lax.dot_general(wg[...].astype(jnp.bfloat16), xv,
                              (((1,), (1,)), ((), ())),
                              preferred_element_type=jnp.float32)
    lgT = lgT + bg[...]
    row = jax.lax.broadcasted_iota(jnp.int32, (NUM_EXPERTS, BM), 0)
    mx = jnp.max(lgT, axis=0, keepdims=True)
    ex = jnp.exp(lgT - mx)
    sm = ex / jnp.sum(ex, axis=0, keepdims=True)
    m1 = jnp.max(sm, axis=0, keepdims=True)
    i1 = jnp.min(jnp.where(sm == m1, row, NUM_EXPERTS), axis=0, keepdims=True)
    sm2 = jnp.where(row == i1, -1.0, sm)
    m2 = jnp.max(sm2, axis=0, keepdims=True)
    i2 = jnp.min(jnp.where(sm2 == m2, row, NUM_EXPERTS), axis=0, keepdims=True)
    den = m1 + m2
    pack = jnp.concatenate(
        [i1.astype(jnp.float32), i2.astype(jnp.float32),
         m1 * (SCALING / den), m2 * (SCALING / den)],
        axis=0)
    packT = jnp.transpose(pack)
    i1c = packT[:, 0:1]
    i2c = packT[:, 1:2]
    w1c = packT[:, 2:3]
    w2c = packT[:, 3:4]
    ecol = (jax.lax.broadcasted_iota(jnp.int32, (BM, ER), 1) // LORA_R
            ).astype(jnp.float32)
    gates = jnp.where(ecol == i1c, w1c, 0.0) + jnp.where(ecol == i2c, w2c, 0.0)
    h = jax.lax.dot_general(xv, a_bf[...],
                            (((1,), (1,)), ((), ())),
                            preferred_element_type=jnp.float32)
    hw = (h * gates).astype(jnp.bfloat16)
    ob[...] = jnp.dot(hw, b2[...], preferred_element_type=jnp.float32)


@jax.jit
def _run(flat, wg, bg, a_view, b2):
    n = flat.shape[0]
    grid = (n // BM,)
    return pl.pallas_call(
        _body,
        grid=grid,
        in_specs=[
            pl.BlockSpec((BM, INPUT_DIM), lambda i: (i, 0)),
            pl.BlockSpec((NUM_EXPERTS, INPUT_DIM), lambda i: (0, 0)),
            pl.BlockSpec((NUM_EXPERTS, 1), lambda i: (0, 0)),
            pl.BlockSpec((ER, INPUT_DIM), lambda i: (0, 0)),
            pl.BlockSpec((ER, OUTPUT_DIM), lambda i: (0, 0)),
        ],
        out_specs=pl.BlockSpec((BM, OUTPUT_DIM), lambda i: (i, 0)),
        out_shape=jax.ShapeDtypeStruct((n, OUTPUT_DIM), jnp.float32),
        scratch_shapes=[pltpu.VMEM((ER, INPUT_DIM), jnp.bfloat16)],
        compiler_params=pltpu.CompilerParams(
            dimension_semantics=("arbitrary",),
        ),
    )(flat, wg, bg, a_view, b2)


def kernel(x, W_gate, b_gate, A, B):
    flat = x.reshape(-1, x.shape[-1])
    bg = b_gate.reshape(NUM_EXPERTS, 1)
    a_view = A.reshape(ER, INPUT_DIM)
    b2 = B.transpose(0, 2, 1).reshape(ER, OUTPUT_DIM).astype(jnp.bfloat16)
    out = _run(flat, W_gate, bg, a_view, b2)
    return out.reshape(x.shape[:-1] + (OUTPUT_DIM,))
